# baseline (speedup 1.0000x reference)
import jax
import jax.numpy as jnp
import numpy as np
from jax import lax
from jax.experimental import pallas as pl
from jax.experimental.pallas import tpu as pltpu

_C = 32
_B = 64
_N = 10000
_NB = 200
_GRID = _N // _NB
_ROWS = _NB * _B


def _gumbel_from_counts(cnt):
    x0 = jnp.zeros_like(cnt)
    x1 = cnt
    ks0 = jnp.uint32(0)
    ks1 = jnp.uint32(42)
    ks2 = ks0 ^ ks1 ^ jnp.uint32(0x1BD11BDA)

    def rotl(x, d):
        return (x << jnp.uint32(d)) | (x >> jnp.uint32(32 - d))

    def rounds(x0, x1, rots):
        for r in rots:
            x0 = x0 + x1
            x1 = rotl(x1, r)
            x1 = x0 ^ x1
        return x0, x1

    ra = (13, 15, 26, 6)
    rb = (17, 29, 16, 24)
    x0 = x0 + ks0
    x1 = x1 + ks1
    x0, x1 = rounds(x0, x1, ra)
    x0 = x0 + ks1
    x1 = x1 + ks2 + jnp.uint32(1)
    x0, x1 = rounds(x0, x1, rb)
    x0 = x0 + ks2
    x1 = x1 + ks0 + jnp.uint32(2)
    x0, x1 = rounds(x0, x1, ra)
    x0 = x0 + ks0
    x1 = x1 + ks1 + jnp.uint32(3)
    x0, x1 = rounds(x0, x1, rb)
    x0 = x0 + ks1
    x1 = x1 + ks2 + jnp.uint32(4)
    x0, x1 = rounds(x0, x1, ra)
    x0 = x0 + ks2
    x1 = x1 + ks0 + jnp.uint32(5)
    bits = x0 ^ x1
    fb = (bits >> jnp.uint32(9)) | jnp.uint32(0x3F800000)
    floats = lax.bitcast_convert_type(fb, jnp.float32) - jnp.float32(1.0)
    tiny = jnp.float32(np.finfo(np.float32).tiny)
    u = jnp.maximum(tiny, floats * (jnp.float32(1.0) - tiny) + tiny)
    return -jnp.log(-jnp.log(u))


def _body(ob_ref, w1_ref, b1_ref, w2_ref, b2_ref, out_ref):
    blk = pl.program_id(0)
    ob = ob_ref[...]
    h = ob[:, :, 2:].reshape(_ROWS, _C)
    hid = jnp.maximum(
        jnp.dot(h, w1_ref[...], preferred_element_type=jnp.float32)
        + b1_ref[...], 0.0)
    logits = (jnp.dot(hid, w2_ref[...], preferred_element_type=jnp.float32)
              + b2_ref[...])
    base = (blk * (_ROWS * _C)).astype(jnp.uint32)
    r = lax.broadcasted_iota(jnp.uint32, (_ROWS, _C), 0)
    c = lax.broadcasted_iota(jnp.uint32, (_ROWS, _C), 1)
    cnt = base + r * jnp.uint32(_C) + c
    noise = _gumbel_from_counts(cnt)
    sampled = jnp.argmax(noise + logits, axis=-1).astype(jnp.int32)
    mask = ob[:, :, 0] == jnp.float32(_C)
    out_ref[...] = jnp.where(mask, sampled.reshape(_NB, _B), 0)


def kernel(ob, edge_index, W1, b1, W2, b2):
    del edge_index
    return pl.pallas_call(
        _body,
        grid=(_GRID,),
        in_specs=[
            pl.BlockSpec((_NB, _B, _C + 2), lambda i: (i, 0, 0)),
            pl.BlockSpec((_C, 64), lambda i: (0, 0)),
            pl.BlockSpec((1, 64), lambda i: (0, 0)),
            pl.BlockSpec((64, _C), lambda i: (0, 0)),
            pl.BlockSpec((1, _C), lambda i: (0, 0)),
        ],
        out_specs=pl.BlockSpec((_NB, _B), lambda i: (i, 0)),
        out_shape=jax.ShapeDtypeStruct((_N, _B), jnp.int32),
        compiler_params=pltpu.CompilerParams(
            dimension_semantics=("arbitrary",)),
    )(ob, W1, b1.reshape(1, 64), W2, b2.reshape(1, _C))
